# baseline (speedup 1.0000x reference)
import functools

import jax
import jax.numpy as jnp
from jax import lax
from jax.experimental import pallas as pl
from jax.experimental.pallas import tpu as pltpu
from jax.experimental.pallas import tpu_sc as plsc

V = 8192
N = 32
SPLIT = 16
CW = V // SPLIT

_NC = 2
_NS = 16


def _gather_body(table_hbm, x_hbm, out_hbm, xv, idxbuf, rows, sem):
  c = lax.axis_index("c")
  s = lax.axis_index("s")
  w = c * _NS + s
  pltpu.sync_copy(x_hbm, xv)
  lanes = lax.iota(jnp.int32, 16)
  lo = xv[pl.ds(0, 16)]
  hi = xv[pl.ds(16, 16)]
  half = jnp.where(jnp.full((16,), c, jnp.int32) == 0, lo, hi)
  tok = jnp.sum(jnp.where(lanes == jnp.full((16,), s, jnp.int32), half, 0))
  flat = jnp.broadcast_to(tok * SPLIT, (16,)) + lanes
  idxbuf[...] = flat
  pltpu.async_copy(table_hbm.at[idxbuf], rows, sem).wait()
  pltpu.sync_copy(rows, out_hbm.at[pl.ds(w * SPLIT, SPLIT)])


@functools.lru_cache(maxsize=1)
def _make_gather():
  return pl.kernel(
      _gather_body,
      mesh=plsc.VectorSubcoreMesh(
          core_axis_name="c", subcore_axis_name="s",
          num_cores=_NC, num_subcores=_NS),
      out_type=jax.ShapeDtypeStruct((N * SPLIT, CW), jnp.float32),
      compiler_params=pltpu.CompilerParams(needs_layout_passes=False),
      scratch_types=[
          pltpu.VMEM((N,), jnp.int32),
          pltpu.VMEM((SPLIT,), jnp.int32),
          pltpu.VMEM((SPLIT, CW), jnp.float32),
          pltpu.SemaphoreType.DMA,
      ],
  )


def _loss_body(y_ref, logits_ref, out_ref):
  l = logits_ref[...]
  m = jnp.max(l, axis=1, keepdims=True)
  ssum = jnp.sum(jnp.exp(l - m), axis=1, keepdims=True)
  lse = m + jnp.log(ssum)
  ids = lax.broadcasted_iota(jnp.int32, (N, V), 1)
  tgt = jnp.sum(jnp.where(ids == y_ref[...], l, 0.0), axis=1, keepdims=True)
  out_ref[0, 0] = jnp.sum(lse - tgt) / N


_loss = pl.pallas_call(
    _loss_body,
    out_shape=jax.ShapeDtypeStruct((1, 1), jnp.float32),
    out_specs=pl.BlockSpec(memory_space=pltpu.SMEM),
)


def kernel(x, y, table):
  xf = x.reshape(N).astype(jnp.int32)
  table2 = table.reshape(V * SPLIT, CW)
  logits = _make_gather()(table2, xf).reshape(N, V)
  loss = _loss(y.reshape(N, 1).astype(jnp.int32), logits)[0, 0]
  return logits, loss


# speedup vs baseline: 12.8837x; 12.8837x over previous
import functools

import jax
import jax.numpy as jnp
from jax import lax
from jax.experimental import pallas as pl
from jax.experimental.pallas import tpu as pltpu
from jax.experimental.pallas import tpu_sc as plsc

V = 8192
N = 32
SPLIT = 16
CW = V // SPLIT

_NC = 2
_NS = 16


def _gather_body(table_hbm, x_hbm, out_hbm, xv, idxbuf, row, sem):
  c = lax.axis_index("c")
  s = lax.axis_index("s")
  w = c * _NS + s
  pltpu.sync_copy(x_hbm, xv)
  lanes = lax.iota(jnp.int32, 16)
  lo = xv[pl.ds(0, 16)]
  hi = xv[pl.ds(16, 16)]
  half = jnp.where(jnp.full((16,), c, jnp.int32) == 0, lo, hi)
  plsc.store_compressed(idxbuf.at[pl.ds(0, 16)], half,
                        mask=lanes == jnp.full((16,), s, jnp.int32))
  pltpu.async_copy(table_hbm.at[idxbuf.at[pl.ds(0, 1)]], row, sem).wait()
  pltpu.sync_copy(row, out_hbm.at[pl.ds(w, 1)])


@functools.lru_cache(maxsize=1)
def _make_gather():
  return pl.kernel(
      _gather_body,
      mesh=plsc.VectorSubcoreMesh(
          core_axis_name="c", subcore_axis_name="s",
          num_cores=_NC, num_subcores=_NS),
      out_type=jax.ShapeDtypeStruct((N, V), jnp.float32),
      compiler_params=pltpu.CompilerParams(needs_layout_passes=False),
      scratch_types=[
          pltpu.VMEM((N,), jnp.int32),
          pltpu.VMEM((16,), jnp.int32),
          pltpu.VMEM((1, V), jnp.float32),
          pltpu.SemaphoreType.DMA,
      ],
  )


def _loss_body(y_ref, logits_ref, out_ref):
  l = logits_ref[...]
  m = jnp.max(l, axis=1, keepdims=True)
  ssum = jnp.sum(jnp.exp(l - m), axis=1, keepdims=True)
  lse = m + jnp.log(ssum)
  ids = lax.broadcasted_iota(jnp.int32, (N, V), 1)
  tgt = jnp.sum(jnp.where(ids == y_ref[...], l, 0.0), axis=1, keepdims=True)
  out_ref[0, 0] = jnp.sum(lse - tgt) / N


_loss = pl.pallas_call(
    _loss_body,
    out_shape=jax.ShapeDtypeStruct((1, 1), jnp.float32),
    out_specs=pl.BlockSpec(memory_space=pltpu.SMEM),
)


def kernel(x, y, table):
  xf = x.reshape(N).astype(jnp.int32)
  logits = _make_gather()(table, xf)
  loss = _loss(y.reshape(N, 1).astype(jnp.int32), logits)[0, 0]
  return logits, loss
